# baseline (speedup 1.0000x reference)
import functools

import jax
import jax.numpy as jnp
from jax import lax
from jax.experimental import pallas as pl
from jax.experimental.pallas import tpu as pltpu
from jax.experimental.pallas import tpu_sc as plsc

_NC = 2
_NS = 16
_NW = _NC * _NS


def _pack_table(table):
    n, d = table.shape
    R = 8000
    h = d // 2

    def body(t_ref, o_ref):
        u = lax.bitcast_convert_type(t_ref[...], jnp.uint32)
        r = (u + jnp.uint32(0x7FFF) + ((u >> 16) & jnp.uint32(1))) >> 16
        w = (r[:, h:] << 16) | r[:, :h]
        o_ref[...] = lax.bitcast_convert_type(w, jnp.int32)

    return pl.pallas_call(
        body,
        grid=(n // R,),
        in_specs=[pl.BlockSpec((R, d), lambda i: (i, 0))],
        out_specs=pl.BlockSpec((R, h), lambda i: (i, 0)),
        out_shape=jax.ShapeDtypeStruct((n, h), jnp.int32),
    )(table)


def _unpack_out(packed):
    n, d = packed.shape
    R = 1024

    def body(p_ref, o_ref):
        u = lax.bitcast_convert_type(p_ref[...], jnp.uint32)
        lo = lax.bitcast_convert_type(u << 16, jnp.float32)
        hi = lax.bitcast_convert_type(u & jnp.uint32(0xFFFF0000),
                                      jnp.float32)
        parts = []
        for g in range(8):
            parts.append(lo[:, 16 * g:16 * g + 16])
            parts.append(hi[:, 16 * g:16 * g + 16])
        o_ref[...] = jnp.concatenate(parts, axis=1)

    return pl.pallas_call(
        body,
        grid=(n // R,),
        in_specs=[pl.BlockSpec((R, d), lambda i: (i, 0))],
        out_specs=pl.BlockSpec((R, 2 * d), lambda i: (i, 0)),
        out_shape=jax.ShapeDtypeStruct((n, 2 * d), jnp.float32),
    )(packed)


@functools.partial(jax.jit, static_argnums=(2, 3, 4, 5))
def _emb_gather(x_flat, table_p, B, C, NBUF, K):
    b_per_w = B // _NW
    n_chunks = b_per_w // C
    assert n_chunks * C == b_per_w
    assert n_chunks % NBUF == 0 and NBUF > K >= 1
    mesh = plsc.VectorSubcoreMesh(core_axis_name="c", subcore_axis_name="s")

    @functools.partial(
        pl.kernel,
        out_type=jax.ShapeDtypeStruct((B, 16), jnp.int32),
        mesh=mesh,
        scratch_types=[
            pltpu.VMEM((NBUF, C), jnp.int32),
            pltpu.VMEM((NBUF, C, 16), jnp.int32),
            pltpu.SemaphoreType.DMA((NBUF,)),
            pltpu.SemaphoreType.DMA((NBUF,)),
        ],
        compiler_params=pltpu.CompilerParams(use_tc_tiling_on_sc=False),
    )
    def k(x_hbm, table_hbm, out_hbm, idx_v, rows_v, s_g, s_st):
        wid = lax.axis_index("s") * _NC + lax.axis_index("c")
        base = wid * b_per_w

        def gather_copy(b):
            return pltpu.make_async_copy(
                table_hbm.at[idx_v.at[b]], rows_v.at[b], s_g.at[b])

        def store_copy(g, b):
            return pltpu.make_async_copy(
                rows_v.at[b],
                out_hbm.at[pl.ds(base + g * C, C)], s_st.at[b])

        def outer(g2, carry):
            for j in range(NBUF):
                g = g2 * NBUF + j

                @pl.when(g >= NBUF)
                def _():
                    store_copy(g - NBUF, j).wait()

                pltpu.sync_copy(x_hbm.at[pl.ds(base + g * C, C)],
                                idx_v.at[j])
                gather_copy(j).start()

                jd = (j - K) % NBUF

                @pl.when(g >= K)
                def _():
                    gather_copy(jd).wait()
                    store_copy(g - K, jd).start()

            return carry

        lax.fori_loop(0, n_chunks // NBUF, outer, 0)

        for c in range(n_chunks - K, n_chunks):
            b = c % NBUF
            gather_copy(b).wait()
            store_copy(c, b).start()
        for c in range(n_chunks - NBUF, n_chunks):
            store_copy(c, c % NBUF).wait()

    return k(x_flat, table_p)


def kernel(x, table):
    B = x.shape[0] * x.shape[1]
    D = table.shape[1]
    table_p = _pack_table(table)
    out_p = _emb_gather(x.reshape(B).astype(jnp.int32), table_p, B,
                        2048, 2, 1)
    return _unpack_out(out_p.reshape(B // 8, 128)).reshape(
        x.shape[0], x.shape[1], D)


# speedup vs baseline: 1.3725x; 1.3725x over previous
import functools

import jax
import jax.numpy as jnp
from jax import lax
from jax.experimental import pallas as pl
from jax.experimental.pallas import tpu as pltpu
from jax.experimental.pallas import tpu_sc as plsc

_NC = 2
_NS = 16
_NW = _NC * _NS


@functools.partial(jax.jit, static_argnums=(2, 3, 4))
def _emb_gather(x_flat, table, B, D, C):
    b_per_w = B // _NW
    n_chunks = b_per_w // C
    assert n_chunks * C == b_per_w and n_chunks >= 2
    mesh = plsc.VectorSubcoreMesh(core_axis_name="c", subcore_axis_name="s")

    @functools.partial(
        pl.kernel,
        out_type=jax.ShapeDtypeStruct((B, D), jnp.float32),
        mesh=mesh,
        scratch_types=[
            pltpu.VMEM((2, C), jnp.int32),
            pltpu.VMEM((2, C, D), jnp.float32),
            pltpu.SemaphoreType.DMA((2,)),
            pltpu.SemaphoreType.DMA((2,)),
            pltpu.SemaphoreType.DMA((2,)),
        ],
        compiler_params=pltpu.CompilerParams(use_tc_tiling_on_sc=False),
    )
    def k2(x_hbm, table_hbm, out_hbm, idx_v, rows_v, s_idx, s_g, s_st):
        wid = lax.axis_index("s") * _NC + lax.axis_index("c")
        base = wid * b_per_w

        def idx_copy(g, b):
            return pltpu.make_async_copy(
                x_hbm.at[pl.ds(base + g * C, C)], idx_v.at[b], s_idx.at[b])

        def gather_copy(b):
            return pltpu.make_async_copy(
                table_hbm.at[idx_v.at[b]], rows_v.at[b], s_g.at[b])

        def store_copy(g, b):
            return pltpu.make_async_copy(
                rows_v.at[b], out_hbm.at[pl.ds(base + g * C, C)], s_st.at[b])

        idx_copy(0, 0).start()
        idx_copy(1, 1).start()
        idx_copy(0, 0).wait()
        gather_copy(0).start()

        def body(g, carry):
            b = g % 2
            pb = 1 - b
            @pl.when(g >= 2)
            def _():
                store_copy(g, b).wait()
            idx_copy(g, b).wait()
            gather_copy(b).start()
            gather_copy(pb).wait()
            store_copy(g - 1, pb).start()

            @pl.when(g + 1 < n_chunks)
            def _():
                idx_copy(g + 1, pb).start()

            return carry

        lax.fori_loop(1, n_chunks, body, 0, unroll=2)

        lb = (n_chunks - 1) % 2
        gather_copy(lb).wait()
        store_copy(n_chunks - 1, lb).start()
        store_copy(n_chunks - 2, 1 - lb).wait()
        store_copy(n_chunks - 1, lb).wait()

    return k2(x_flat, table)


def kernel(x, table):
    B = x.shape[0] * x.shape[1]
    D = table.shape[1]
    out = _emb_gather(x.reshape(B).astype(jnp.int32), table, B, D, 1600)
    return out.reshape(x.shape[0], x.shape[1], D)
